# baseline (speedup 1.0000x reference)
import jax
import jax.numpy as jnp
from jax.experimental import pallas as pl
from jax.experimental.pallas import tpu as pltpu

_N = 100000
_H = 128
_CHUNKS = 8
_ROWS = _N // _CHUNKS


def _copy_body(in_hbm, out_hbm, sems):
    for c in range(_CHUNKS):
        pltpu.make_async_copy(
            in_hbm.at[pl.ds(c * _ROWS, _ROWS), :],
            out_hbm.at[pl.ds(c * _ROWS, _ROWS), :],
            sems.at[c],
        ).start()
    for c in range(_CHUNKS):
        pltpu.make_async_copy(
            in_hbm.at[pl.ds(c * _ROWS, _ROWS), :],
            out_hbm.at[pl.ds(c * _ROWS, _ROWS), :],
            sems.at[c],
        ).wait()


def kernel(table):
    return pl.pallas_call(
        _copy_body,
        in_specs=[pl.BlockSpec(memory_space=pl.ANY)],
        out_specs=pl.BlockSpec(memory_space=pl.ANY),
        out_shape=jax.ShapeDtypeStruct((_N, _H), table.dtype),
        scratch_shapes=[pltpu.SemaphoreType.DMA((_CHUNKS,))],
    )(table)


# speedup vs baseline: 28.0143x; 28.0143x over previous
import functools

import jax
import jax.numpy as jnp
from jax import lax
from jax.experimental import pallas as pl
from jax.experimental.pallas import tpu as pltpu
from jax.experimental.pallas import tpu_sc as plsc

_N = 100000
_H = 128
_WORDS = _N * _H
_NW = 32
_PER_W = _WORDS // _NW
_CHUNK = 20_000
_NCHUNK = _PER_W // _CHUNK


@functools.partial(
    pl.kernel,
    mesh=plsc.VectorSubcoreMesh(core_axis_name="c", subcore_axis_name="s"),
    out_type=jax.ShapeDtypeStruct((_WORDS,), jnp.float32),
    scratch_types=[
        pltpu.VMEM((_CHUNK,), jnp.float32),
        pltpu.VMEM((_CHUNK,), jnp.float32),
        pltpu.SemaphoreType.DMA,
        pltpu.SemaphoreType.DMA,
        pltpu.SemaphoreType.DMA,
        pltpu.SemaphoreType.DMA,
    ],
)
def _sc_copy(tab_hbm, out_hbm, buf0, buf1, si0, si1, so0, so1):
    wid = lax.axis_index("s") * 2 + lax.axis_index("c")
    base = wid * _PER_W
    bufs = (buf0, buf1)
    sin = (si0, si1)
    sout = (so0, so1)

    def in_copy(i):
        return pltpu.async_copy(
            tab_hbm.at[pl.ds(base + i * _CHUNK, _CHUNK)], bufs[i % 2], sin[i % 2]
        )

    def out_copy(i):
        return pltpu.async_copy(
            bufs[i % 2], out_hbm.at[pl.ds(base + i * _CHUNK, _CHUNK)], sout[i % 2]
        )

    hin = [None, None]
    hout = [None, None]
    hin[0] = in_copy(0)
    for i in range(_NCHUNK):
        b = i % 2
        if i + 1 < _NCHUNK:
            b2 = (i + 1) % 2
            if hout[b2] is not None:
                hout[b2].wait()
            hin[b2] = in_copy(i + 1)
        hin[b].wait()
        hout[b] = out_copy(i)
    hout[(_NCHUNK - 2) % 2].wait()
    hout[(_NCHUNK - 1) % 2].wait()


def kernel(table):
    flat = table.reshape(_WORDS)
    return _sc_copy(flat).reshape(_N, _H)


# speedup vs baseline: 36.9903x; 1.3204x over previous
import jax
import jax.numpy as jnp
from jax.experimental import pallas as pl
from jax.experimental.pallas import tpu as pltpu

_N = 100000
_H = 128
_CHUNK = 5000
_NCHUNK = _N // _CHUNK


def _copy_body(in_hbm, out_hbm, buf0, buf1, si0, si1, so0, so1):
    bufs = (buf0, buf1)
    sin = (si0, si1)
    sout = (so0, so1)

    def in_copy(i):
        return pltpu.make_async_copy(
            in_hbm.at[pl.ds(i * _CHUNK, _CHUNK), :], bufs[i % 2], sin[i % 2]
        )

    def out_copy(i):
        return pltpu.make_async_copy(
            bufs[i % 2], out_hbm.at[pl.ds(i * _CHUNK, _CHUNK), :], sout[i % 2]
        )

    hin = [None, None]
    hout = [None, None]
    hin[0] = in_copy(0)
    hin[0].start()
    for i in range(_NCHUNK):
        b = i % 2
        if i + 1 < _NCHUNK:
            b2 = (i + 1) % 2
            if hout[b2] is not None:
                hout[b2].wait()
            hin[b2] = in_copy(i + 1)
            hin[b2].start()
        hin[b].wait()
        hout[b] = out_copy(i)
        hout[b].start()
    hout[(_NCHUNK - 2) % 2].wait()
    hout[(_NCHUNK - 1) % 2].wait()


def kernel(table):
    return pl.pallas_call(
        _copy_body,
        in_specs=[pl.BlockSpec(memory_space=pl.ANY)],
        out_specs=pl.BlockSpec(memory_space=pl.ANY),
        out_shape=jax.ShapeDtypeStruct((_N, _H), table.dtype),
        scratch_shapes=[
            pltpu.VMEM((_CHUNK, _H), jnp.float32),
            pltpu.VMEM((_CHUNK, _H), jnp.float32),
            pltpu.SemaphoreType.DMA,
            pltpu.SemaphoreType.DMA,
            pltpu.SemaphoreType.DMA,
            pltpu.SemaphoreType.DMA,
        ],
    )(table)


# speedup vs baseline: 42.4554x; 1.1477x over previous
import jax
import jax.numpy as jnp
from jax.experimental import pallas as pl

_N = 100000
_H = 128
_BLOCK = 4000


def _copy_body(in_ref, out_ref):
    out_ref[...] = in_ref[...]


def kernel(table):
    grid = (_N // _BLOCK,)
    return pl.pallas_call(
        _copy_body,
        grid=grid,
        in_specs=[pl.BlockSpec((_BLOCK, _H), lambda i: (i, 0))],
        out_specs=pl.BlockSpec((_BLOCK, _H), lambda i: (i, 0)),
        out_shape=jax.ShapeDtypeStruct((_N, _H), table.dtype),
    )(table)


# speedup vs baseline: 49.2446x; 1.1599x over previous
import jax
import jax.numpy as jnp
from jax.experimental import pallas as pl

_N = 100000
_H = 128
_BLOCK = 20000


def _copy_body(in_ref, out_ref):
    out_ref[...] = in_ref[...]


def kernel(table):
    grid = (_N // _BLOCK,)
    return pl.pallas_call(
        _copy_body,
        grid=grid,
        in_specs=[pl.BlockSpec((_BLOCK, _H), lambda i: (i, 0))],
        out_specs=pl.BlockSpec((_BLOCK, _H), lambda i: (i, 0)),
        out_shape=jax.ShapeDtypeStruct((_N, _H), table.dtype),
    )(table)
